# baseline (speedup 1.0000x reference)
import functools

import jax
import jax.numpy as jnp
from jax import lax
from jax.experimental import pallas as pl
from jax.experimental.pallas import tpu as pltpu
from jax.experimental.pallas import tpu_sc as plsc

N = 10000
E = 320000
D = 128
H = 64
G = 128

NTILES = 32
ROWS_PER_TILE = 640
NPAD = NTILES * ROWS_PER_TILE // 2
CHUNK = 128
NBUF = 2
ZROWS = 640
NCHUNK = 80
EPAD = NTILES * NCHUNK * CHUNK

_f32 = jnp.float32


def _sc_segsum_body(u_hbm, src_hbm, dst_hbm, out_hbm, idx_s, idx_d, rows,
                    zbuf, acc, sem):
    c = lax.axis_index("c")
    s = lax.axis_index("s")
    wid = c * 16 + s

    def _zrow(i, carry):
        for k in range(4):
            zbuf[i, pl.ds(k * 16, 16)] = jnp.zeros((16,), _f32)
        return carry

    lax.fori_loop(0, ZROWS, _zrow, 0)
    pltpu.sync_copy(zbuf, acc.at[pl.ds(s * ROWS_PER_TILE, ROWS_PER_TILE)])

    pltpu.sync_copy(src_hbm.at[wid], idx_s)
    pltpu.sync_copy(dst_hbm.at[wid], idx_d)
    plsc.subcore_barrier()

    def _chunk(j, carry):
        pltpu.async_copy(u_hbm.at[idx_s.at[j]], rows, sem).wait()
        pltpu.sync_copy(rows, acc.at[idx_d.at[j]], add=True)
        return carry

    lax.fori_loop(0, NCHUNK, _chunk, 0)
    plsc.subcore_barrier()

    sl = pl.ds(s * ROWS_PER_TILE, ROWS_PER_TILE)
    pltpu.sync_copy(acc.at[sl], out_hbm.at[c].at[sl])


@jax.jit
def _sc_segsum(u, src_r, dst_r):
    mesh = plsc.VectorSubcoreMesh(core_axis_name="c", subcore_axis_name="s")
    kfn = pl.kernel(
        _sc_segsum_body,
        out_type=jax.ShapeDtypeStruct((2, NPAD, H), _f32),
        mesh=mesh,
        scratch_types=[
            pltpu.VMEM((NCHUNK, CHUNK), jnp.int32),
            pltpu.VMEM((NCHUNK, CHUNK), jnp.int32),
            pltpu.VMEM((CHUNK, H), _f32),
            pltpu.VMEM((ROWS_PER_TILE, H), _f32),
            pltpu.VMEM_SHARED((NPAD, H), _f32),
            pltpu.SemaphoreType.DMA,
        ],
        compiler_params=pltpu.CompilerParams(use_tc_tiling_on_sc=False),
    )
    return kfn(u, src_r, dst_r)


def _group_mask(batch_row):
    g_iota = lax.broadcasted_iota(jnp.int32, (G, NPAD), 0)
    return (batch_row[None, :] == g_iota).astype(_f32)


def _masked_bn(z, nmask, g, b):
    mu = jnp.sum(z * nmask, axis=0, keepdims=True) * (1.0 / N)
    zc = z - mu
    var = jnp.sum(zc * zc * nmask, axis=0, keepdims=True) * (1.0 / N)
    return g * zc * lax.rsqrt(var + 1e-5) + b


def _tc_first_body(x_ref, batch_ref, w1_ref, u_ref, pool_ref):
    x = x_ref[...]
    u_ref[...] = jnp.dot(x, w1_ref[...], preferred_element_type=_f32, precision=lax.Precision.HIGHEST)
    pool_ref[...] = jnp.dot(_group_mask(batch_ref[...]), x,
                            preferred_element_type=_f32, precision=lax.Precision.HIGHEST)


def _conv_tail(u_ref, parts_ref, b1, g1, be1, w2, b2, g2, be2):
    nmask = (lax.broadcasted_iota(jnp.int32, (NPAD, 1), 0) < N).astype(_f32)
    z = u_ref[...] + parts_ref[0] + parts_ref[1] + b1[...]
    z = _masked_bn(z, nmask, g1[...], be1[...])
    r = jnp.maximum(z, 0.0) * nmask
    v = jnp.dot(r, w2[...], preferred_element_type=_f32, precision=lax.Precision.HIGHEST) + b2[...]
    v = _masked_bn(v, nmask, g2[...], be2[...])
    return jnp.maximum(v, 0.0) * nmask


def _tc_mid_body(u_ref, parts_ref, b1, g1, be1, w2, b2, g2, be2, batch_ref,
                 w1n, un_ref, pool_ref):
    h = _conv_tail(u_ref, parts_ref, b1, g1, be1, w2, b2, g2, be2)
    pool_ref[...] = jnp.dot(_group_mask(batch_ref[...]), h,
                            preferred_element_type=_f32, precision=lax.Precision.HIGHEST)
    un_ref[...] = jnp.dot(h, w1n[...], preferred_element_type=_f32, precision=lax.Precision.HIGHEST)


def _tc_last_body(u_ref, parts_ref, b1, g1, be1, w2, b2, g2, be2, batch_ref,
                  p0, p1, p2, lw0, lb0, lw1, lb1, y_ref):
    h = _conv_tail(u_ref, parts_ref, b1, g1, be1, w2, b2, g2, be2)
    p3 = jnp.dot(_group_mask(batch_ref[...]), h, preferred_element_type=_f32, precision=lax.Precision.HIGHEST)
    hcat = jnp.concatenate([p0[...], p1[...], p2[...], p3], axis=1)
    y = jnp.dot(hcat, lw0[...], preferred_element_type=_f32, precision=lax.Precision.HIGHEST) + lb0[...]
    y = jnp.maximum(y, 0.0)
    y_ref[...] = jnp.dot(y, lw1[...], preferred_element_type=_f32, precision=lax.Precision.HIGHEST) + lb1[...]


def _tc_first(x_pad, batch_pad, w1):
    return pl.pallas_call(
        _tc_first_body,
        out_shape=[jax.ShapeDtypeStruct((NPAD, H), _f32),
                   jax.ShapeDtypeStruct((G, D), _f32)],
    )(x_pad, batch_pad, w1)


def _tc_mid(u, parts, cp, ob, batch_pad, w1n):
    return pl.pallas_call(
        _tc_mid_body,
        out_shape=[jax.ShapeDtypeStruct((NPAD, H), _f32),
                   jax.ShapeDtypeStruct((G, H), _f32)],
    )(u, parts, cp["b1"], cp["bng"], cp["bnb"], cp["w2"], cp["b2"],
      ob["g"], ob["b"], batch_pad, w1n)


def _tc_last(u, parts, cp, ob, batch_pad, pools, lin0, lin1):
    return pl.pallas_call(
        _tc_last_body,
        out_shape=jax.ShapeDtypeStruct((G, 1), _f32),
    )(u, parts, cp["b1"], cp["bng"], cp["bnb"], cp["w2"], cp["b2"],
      ob["g"], ob["b"], batch_pad, pools[0], pools[1], pools[2],
      lin0["w"], lin0["b"], lin1["w"], lin1["b"])


def kernel(x, edge_index, batch, params):
    x_pad = jnp.zeros((NPAD, D), _f32).at[:N].set(x)
    batch_pad = jnp.zeros((NPAD,), jnp.int32).at[:N].set(batch)
    pad = EPAD - E
    src_r = jnp.concatenate(
        [edge_index[0], jnp.full((pad,), N, jnp.int32)]).reshape(
            NTILES, NCHUNK, CHUNK)
    dst_r = jnp.concatenate(
        [edge_index[1], jnp.full((pad,), N, jnp.int32)]).reshape(
            NTILES, NCHUNK, CHUNK)

    convs = params["convs"]
    obn = params["obn"]
    row = lambda v: v.reshape(1, -1)

    def conv_params(i):
        cp = convs[i]
        return ({"b1": row(cp["b1"]), "bng": row(cp["bng"]),
                 "bnb": row(cp["bnb"]), "w2": cp["w2"], "b2": row(cp["b2"])},
                {"g": row(obn[i]["g"]), "b": row(obn[i]["b"])})

    u0, pool0 = _tc_first(x_pad, batch_pad, convs[0]["w1"])
    parts0 = _sc_segsum(u0, src_r, dst_r)
    cp0, ob0 = conv_params(0)
    u1, pool1 = _tc_mid(u0, parts0, cp0, ob0, batch_pad, convs[1]["w1"])
    parts1 = _sc_segsum(u1, src_r, dst_r)
    cp1, ob1 = conv_params(1)
    u2, pool2 = _tc_mid(u1, parts1, cp1, ob1, batch_pad, convs[2]["w1"])
    parts2 = _sc_segsum(u2, src_r, dst_r)
    cp2, ob2 = conv_params(2)
    lin0 = {"w": params["lins"][0]["w"], "b": row(params["lins"][0]["b"])}
    lin1 = {"w": params["lins"][1]["w"], "b": row(params["lins"][1]["b"])}
    y = _tc_last(u2, parts2, cp2, ob2, batch_pad,
                 [pool0, pool1, pool2], lin0, lin1)
    return y.reshape(-1)


# speedup vs baseline: 1.4303x; 1.4303x over previous
import functools

import jax
import jax.numpy as jnp
from jax import lax
from jax.experimental import pallas as pl
from jax.experimental.pallas import tpu as pltpu
from jax.experimental.pallas import tpu_sc as plsc

N = 10000
E = 320000
D = 128
H = 64
G = 128

NTILES = 32
ROWS_PER_TILE = 640
NPAD = NTILES * ROWS_PER_TILE // 2
CHUNK = 128
NBUF = 2
ZROWS = 640
NCHUNK = 79
EPAD = NTILES * NCHUNK * CHUNK

_f32 = jnp.float32


def _sc_segsum_body(u_hbm, src_hbm, dst_hbm, out_hbm, idx_s, idx_d, rows,
                    zbuf, acc, sem):
    c = lax.axis_index("c")
    s = lax.axis_index("s")
    wid = c * 16 + s

    def _zrow(i, carry):
        for k in range(4):
            zbuf[i, pl.ds(k * 16, 16)] = jnp.zeros((16,), _f32)
        return carry

    lax.fori_loop(0, ZROWS, _zrow, 0)
    pltpu.sync_copy(zbuf, acc.at[pl.ds(s * ROWS_PER_TILE, ROWS_PER_TILE)])

    pltpu.sync_copy(src_hbm.at[wid], idx_s)
    pltpu.sync_copy(dst_hbm.at[wid], idx_d)
    plsc.subcore_barrier()

    def _chunk(j, carry):
        pltpu.async_copy(u_hbm.at[idx_s.at[j]], rows, sem).wait()
        pltpu.sync_copy(rows, acc.at[idx_d.at[j]], add=True)
        return carry

    lax.fori_loop(0, NCHUNK, _chunk, 0)
    plsc.subcore_barrier()

    sl = pl.ds(s * ROWS_PER_TILE, ROWS_PER_TILE)
    pltpu.sync_copy(acc.at[sl], out_hbm.at[c].at[sl])


@jax.jit
def _sc_segsum(u, src_r, dst_r):
    mesh = plsc.VectorSubcoreMesh(core_axis_name="c", subcore_axis_name="s")
    kfn = pl.kernel(
        _sc_segsum_body,
        out_type=jax.ShapeDtypeStruct((2, NPAD, H), _f32),
        mesh=mesh,
        scratch_types=[
            pltpu.VMEM((NCHUNK, CHUNK), jnp.int32),
            pltpu.VMEM((NCHUNK, CHUNK), jnp.int32),
            pltpu.VMEM((CHUNK, H), _f32),
            pltpu.VMEM((ROWS_PER_TILE, H), _f32),
            pltpu.VMEM_SHARED((NPAD, H), _f32),
            pltpu.SemaphoreType.DMA,
        ],
        compiler_params=pltpu.CompilerParams(use_tc_tiling_on_sc=False),
    )
    return kfn(u, src_r, dst_r)


def _group_mask(batch_row):
    g_iota = lax.broadcasted_iota(jnp.int32, (G, NPAD), 0)
    return (batch_row[None, :] == g_iota).astype(_f32)


def _masked_bn(z, nmask, g, b):
    mu = jnp.sum(z * nmask, axis=0, keepdims=True) * (1.0 / N)
    zc = z - mu
    var = jnp.sum(zc * zc * nmask, axis=0, keepdims=True) * (1.0 / N)
    return g * zc * lax.rsqrt(var + 1e-5) + b


def _tc_first_body(x_ref, batch_ref, w1_ref, u_ref, pool_ref):
    x = x_ref[...]
    u_ref[...] = jnp.dot(x, w1_ref[...], preferred_element_type=_f32, precision=lax.Precision.HIGHEST)
    pool_ref[...] = jnp.dot(_group_mask(batch_ref[...]), x,
                            preferred_element_type=_f32, precision=lax.Precision.HIGHEST)


def _conv_tail(u_ref, parts_ref, b1, g1, be1, w2, b2, g2, be2):
    nmask = (lax.broadcasted_iota(jnp.int32, (NPAD, 1), 0) < N).astype(_f32)
    z = u_ref[...] + parts_ref[0] + parts_ref[1] + b1[...]
    z = _masked_bn(z, nmask, g1[...], be1[...])
    r = jnp.maximum(z, 0.0) * nmask
    v = jnp.dot(r, w2[...], preferred_element_type=_f32, precision=lax.Precision.HIGHEST) + b2[...]
    v = _masked_bn(v, nmask, g2[...], be2[...])
    return jnp.maximum(v, 0.0) * nmask


def _tc_mid_body(u_ref, parts_ref, b1, g1, be1, w2, b2, g2, be2, batch_ref,
                 w1n, un_ref, pool_ref):
    h = _conv_tail(u_ref, parts_ref, b1, g1, be1, w2, b2, g2, be2)
    pool_ref[...] = jnp.dot(_group_mask(batch_ref[...]), h,
                            preferred_element_type=_f32, precision=lax.Precision.HIGHEST)
    un_ref[...] = jnp.dot(h, w1n[...], preferred_element_type=_f32, precision=lax.Precision.HIGHEST)


def _tc_last_body(u_ref, parts_ref, b1, g1, be1, w2, b2, g2, be2, batch_ref,
                  p0, p1, p2, lw0, lb0, lw1, lb1, y_ref):
    h = _conv_tail(u_ref, parts_ref, b1, g1, be1, w2, b2, g2, be2)
    p3 = jnp.dot(_group_mask(batch_ref[...]), h, preferred_element_type=_f32, precision=lax.Precision.HIGHEST)
    hcat = jnp.concatenate([p0[...], p1[...], p2[...], p3], axis=1)
    y = jnp.dot(hcat, lw0[...], preferred_element_type=_f32, precision=lax.Precision.HIGHEST) + lb0[...]
    y = jnp.maximum(y, 0.0)
    y_ref[...] = jnp.dot(y, lw1[...], preferred_element_type=_f32, precision=lax.Precision.HIGHEST) + lb1[...]


def _tc_first(x_pad, batch_pad, w1):
    return pl.pallas_call(
        _tc_first_body,
        out_shape=[jax.ShapeDtypeStruct((NPAD, H), _f32),
                   jax.ShapeDtypeStruct((G, D), _f32)],
    )(x_pad, batch_pad, w1)


def _tc_mid(u, parts, cp, ob, batch_pad, w1n):
    return pl.pallas_call(
        _tc_mid_body,
        out_shape=[jax.ShapeDtypeStruct((NPAD, H), _f32),
                   jax.ShapeDtypeStruct((G, H), _f32)],
    )(u, parts, cp["b1"], cp["bng"], cp["bnb"], cp["w2"], cp["b2"],
      ob["g"], ob["b"], batch_pad, w1n)


def _tc_last(u, parts, cp, ob, batch_pad, pools, lin0, lin1):
    return pl.pallas_call(
        _tc_last_body,
        out_shape=jax.ShapeDtypeStruct((G, 1), _f32),
    )(u, parts, cp["b1"], cp["bng"], cp["bnb"], cp["w2"], cp["b2"],
      ob["g"], ob["b"], batch_pad, pools[0], pools[1], pools[2],
      lin0["w"], lin0["b"], lin1["w"], lin1["b"])


def kernel(x, edge_index, batch, params):
    x_pad = jnp.zeros((NPAD, D), _f32).at[:N].set(x)
    batch_pad = jnp.zeros((NPAD,), jnp.int32).at[:N].set(batch)
    pad = EPAD - E
    src_r = jnp.concatenate(
        [edge_index[0], jnp.full((pad,), N, jnp.int32)]).reshape(
            NTILES, NCHUNK, CHUNK)
    dst_r = jnp.concatenate(
        [edge_index[1], jnp.full((pad,), N, jnp.int32)]).reshape(
            NTILES, NCHUNK, CHUNK)

    convs = params["convs"]
    obn = params["obn"]
    row = lambda v: v.reshape(1, -1)

    def conv_params(i):
        cp = convs[i]
        return ({"b1": row(cp["b1"]), "bng": row(cp["bng"]),
                 "bnb": row(cp["bnb"]), "w2": cp["w2"], "b2": row(cp["b2"])},
                {"g": row(obn[i]["g"]), "b": row(obn[i]["b"])})

    u0, pool0 = _tc_first(x_pad, batch_pad, convs[0]["w1"])
    parts0 = _sc_segsum(u0, src_r, dst_r)
    cp0, ob0 = conv_params(0)
    u1, pool1 = _tc_mid(u0, parts0, cp0, ob0, batch_pad, convs[1]["w1"])
    parts1 = _sc_segsum(u1, src_r, dst_r)
    cp1, ob1 = conv_params(1)
    u2, pool2 = _tc_mid(u1, parts1, cp1, ob1, batch_pad, convs[2]["w1"])
    parts2 = _sc_segsum(u2, src_r, dst_r)
    cp2, ob2 = conv_params(2)
    lin0 = {"w": params["lins"][0]["w"], "b": row(params["lins"][0]["b"])}
    lin1 = {"w": params["lins"][1]["w"], "b": row(params["lins"][1]["b"])}
    y = _tc_last(u2, parts2, cp2, ob2, batch_pad,
                 [pool0, pool1, pool2], lin0, lin1)
    return y.reshape(-1)


# speedup vs baseline: 2.2201x; 1.5521x over previous
import functools

import jax
import jax.numpy as jnp
from jax import lax
from jax.experimental import pallas as pl
from jax.experimental.pallas import tpu as pltpu
from jax.experimental.pallas import tpu_sc as plsc

N = 10000
E = 320000
D = 128
H = 64
G = 128

NTILES = 32
ROWS_PER_TILE = 640
NPAD = NTILES * ROWS_PER_TILE // 2
CHUNK = 128
NBUF = 2
ZROWS = 640
NCHUNK = 80
EPAD = NTILES * NCHUNK * CHUNK

_f32 = jnp.float32


def _sc_segsum_body(u_hbm, src_hbm, dst_hbm, out_hbm, idx_s, idx_d, rows,
                    zbuf, acc, sem):
    c = lax.axis_index("c")
    s = lax.axis_index("s")
    wid = c * 16 + s

    def _zrow(i, carry):
        for k in range(4):
            zbuf[i, pl.ds(k * 16, 16)] = jnp.zeros((16,), _f32)
        return carry

    lax.fori_loop(0, ZROWS, _zrow, 0)
    pltpu.sync_copy(zbuf, acc.at[pl.ds(s * ROWS_PER_TILE, ROWS_PER_TILE)])

    pltpu.sync_copy(src_hbm.at[wid], idx_s)
    pltpu.sync_copy(dst_hbm.at[wid], idx_d)
    plsc.subcore_barrier()

    def _chunk(j, carry):
        pltpu.async_copy(u_hbm.at[idx_s.at[j]], rows, sem).wait()
        pltpu.sync_copy(rows, acc.at[idx_d.at[j]], add=True)
        return carry

    lax.fori_loop(0, NCHUNK, _chunk, 0)
    plsc.subcore_barrier()

    sl = pl.ds(s * ROWS_PER_TILE, ROWS_PER_TILE)
    pltpu.sync_copy(acc.at[sl], out_hbm.at[c].at[sl])


@jax.jit
def _sc_segsum(u, src_r, dst_r):
    mesh = plsc.VectorSubcoreMesh(core_axis_name="c", subcore_axis_name="s")
    kfn = pl.kernel(
        _sc_segsum_body,
        out_type=jax.ShapeDtypeStruct((2, NPAD, H), _f32),
        mesh=mesh,
        scratch_types=[
            pltpu.VMEM((NCHUNK, CHUNK), jnp.int32),
            pltpu.VMEM((NCHUNK, CHUNK), jnp.int32),
            pltpu.VMEM((CHUNK, H), _f32),
            pltpu.VMEM((ROWS_PER_TILE, H), _f32),
            pltpu.VMEM_SHARED((NPAD, H), _f32),
            pltpu.SemaphoreType.DMA,
        ],
        compiler_params=pltpu.CompilerParams(use_tc_tiling_on_sc=False),
    )
    return kfn(u, src_r, dst_r)


def _group_mask(batch_row):
    g_iota = lax.broadcasted_iota(jnp.int32, (G, NPAD), 0)
    return (batch_row[None, :] == g_iota).astype(_f32)


def _masked_bn(z, nmask, g, b):
    mu = jnp.sum(z * nmask, axis=0, keepdims=True) * (1.0 / N)
    zc = z - mu
    var = jnp.sum(zc * zc * nmask, axis=0, keepdims=True) * (1.0 / N)
    return g * zc * lax.rsqrt(var + 1e-5) + b


def _tc_first_body(x_ref, batch_ref, w1_ref, u_ref, pool_ref):
    x = x_ref[...]
    u_ref[...] = jnp.dot(x, w1_ref[...], preferred_element_type=_f32, precision=lax.Precision.HIGHEST)
    pool_ref[...] = jnp.dot(_group_mask(batch_ref[...]), x,
                            preferred_element_type=_f32, precision=lax.Precision.HIGHEST)


def _conv_tail(u_ref, parts_ref, b1, g1, be1, w2, b2, g2, be2):
    nmask = (lax.broadcasted_iota(jnp.int32, (NPAD, 1), 0) < N).astype(_f32)
    z = u_ref[...] + parts_ref[0] + parts_ref[1] + b1[...]
    z = _masked_bn(z, nmask, g1[...], be1[...])
    r = jnp.maximum(z, 0.0) * nmask
    v = jnp.dot(r, w2[...], preferred_element_type=_f32, precision=lax.Precision.HIGHEST) + b2[...]
    v = _masked_bn(v, nmask, g2[...], be2[...])
    return jnp.maximum(v, 0.0) * nmask


def _tc_mid_body(u_ref, parts_ref, b1, g1, be1, w2, b2, g2, be2, batch_ref,
                 w1n, un_ref, pool_ref):
    h = _conv_tail(u_ref, parts_ref, b1, g1, be1, w2, b2, g2, be2)
    pool_ref[...] = jnp.dot(_group_mask(batch_ref[...]), h,
                            preferred_element_type=_f32, precision=lax.Precision.HIGHEST)
    un_ref[...] = jnp.dot(h, w1n[...], preferred_element_type=_f32, precision=lax.Precision.HIGHEST)


def _tc_last_body(u_ref, parts_ref, b1, g1, be1, w2, b2, g2, be2, batch_ref,
                  p0, p1, p2, lw0, lb0, lw1, lb1, y_ref):
    h = _conv_tail(u_ref, parts_ref, b1, g1, be1, w2, b2, g2, be2)
    p3 = jnp.dot(_group_mask(batch_ref[...]), h, preferred_element_type=_f32, precision=lax.Precision.HIGHEST)
    hcat = jnp.concatenate([p0[...], p1[...], p2[...], p3], axis=1)
    y = jnp.dot(hcat, lw0[...], preferred_element_type=_f32, precision=lax.Precision.HIGHEST) + lb0[...]
    y = jnp.maximum(y, 0.0)
    y_ref[...] = jnp.dot(y, lw1[...], preferred_element_type=_f32, precision=lax.Precision.HIGHEST) + lb1[...]


def _tc_first(x_pad, batch_pad, w1):
    return pl.pallas_call(
        _tc_first_body,
        out_shape=[jax.ShapeDtypeStruct((NPAD, H), _f32),
                   jax.ShapeDtypeStruct((G, D), _f32)],
    )(x_pad, batch_pad, w1)


def _tc_mid(u, parts, cp, ob, batch_pad, w1n):
    return pl.pallas_call(
        _tc_mid_body,
        out_shape=[jax.ShapeDtypeStruct((NPAD, H), _f32),
                   jax.ShapeDtypeStruct((G, H), _f32)],
    )(u, parts, cp["b1"], cp["bng"], cp["bnb"], cp["w2"], cp["b2"],
      ob["g"], ob["b"], batch_pad, w1n)


def _tc_last(u, parts, cp, ob, batch_pad, pools, lin0, lin1):
    return pl.pallas_call(
        _tc_last_body,
        out_shape=jax.ShapeDtypeStruct((G, 1), _f32),
    )(u, parts, cp["b1"], cp["bng"], cp["bnb"], cp["w2"], cp["b2"],
      ob["g"], ob["b"], batch_pad, pools[0], pools[1], pools[2],
      lin0["w"], lin0["b"], lin1["w"], lin1["b"])


def kernel(x, edge_index, batch, params):
    x_pad = jnp.zeros((NPAD, D), _f32).at[:N].set(x)
    batch_pad = jnp.zeros((NPAD,), jnp.int32).at[:N].set(batch)
    pad_pt = NCHUNK * CHUNK - E // NTILES
    fake = jnp.broadcast_to(
        (N + jnp.arange(pad_pt, dtype=jnp.int32))[None], (NTILES, pad_pt))
    src_r = jnp.concatenate(
        [edge_index[0].reshape(NTILES, -1), fake], axis=1).reshape(
            NTILES, NCHUNK, CHUNK)
    dst_r = jnp.concatenate(
        [edge_index[1].reshape(NTILES, -1), fake], axis=1).reshape(
            NTILES, NCHUNK, CHUNK)

    convs = params["convs"]
    obn = params["obn"]
    row = lambda v: v.reshape(1, -1)

    def conv_params(i):
        cp = convs[i]
        return ({"b1": row(cp["b1"]), "bng": row(cp["bng"]),
                 "bnb": row(cp["bnb"]), "w2": cp["w2"], "b2": row(cp["b2"])},
                {"g": row(obn[i]["g"]), "b": row(obn[i]["b"])})

    u0, pool0 = _tc_first(x_pad, batch_pad, convs[0]["w1"])
    parts0 = _sc_segsum(u0, src_r, dst_r)
    cp0, ob0 = conv_params(0)
    u1, pool1 = _tc_mid(u0, parts0, cp0, ob0, batch_pad, convs[1]["w1"])
    parts1 = _sc_segsum(u1, src_r, dst_r)
    cp1, ob1 = conv_params(1)
    u2, pool2 = _tc_mid(u1, parts1, cp1, ob1, batch_pad, convs[2]["w1"])
    parts2 = _sc_segsum(u2, src_r, dst_r)
    cp2, ob2 = conv_params(2)
    lin0 = {"w": params["lins"][0]["w"], "b": row(params["lins"][0]["b"])}
    lin1 = {"w": params["lins"][1]["w"], "b": row(params["lins"][1]["b"])}
    y = _tc_last(u2, parts2, cp2, ob2, batch_pad,
                 [pool0, pool1, pool2], lin0, lin1)
    return y.reshape(-1)
